# baseline (speedup 1.0000x reference)
import math

import jax
import jax.numpy as jnp
from jax import lax
from jax.experimental import pallas as pl
from jax.experimental.pallas import tpu as pltpu
from jax.experimental.pallas import tpu_sc as plsc

_NUM_TOKENS = 2048
_MODEL_DIM = 1024
_NUM_EXPERTS = 64
_CAPACITY = int(2 * math.ceil(_NUM_TOKENS / (_NUM_EXPERTS // 4)) * 1.0)
_CUMSUM_BLK = 128
_WRITE_BLK = 128

_NW = 32
_TPW = _NUM_TOKENS // _NW
_ROWS = _NUM_TOKENS * _NUM_EXPERTS
_RPW = _ROWS // _NW
_ZROWS = 128


def _routing_kernel(x_ref, w_ref, fsel_ref, gate_ref, laux_ref):
    x = x_ref[...]
    w = w_ref[...]
    logits = lax.dot_general(
        x, w, (((1,), (1,)), ((), ())), preferred_element_type=jnp.float32
    )
    col = lax.broadcasted_iota(jnp.int32, (_NUM_TOKENS, _NUM_EXPERTS), 1)
    logits = jnp.where(col == 0, jnp.float32(-1000000000.0), logits)
    m = jnp.max(logits, axis=1, keepdims=True)
    p = jnp.exp(logits - m)
    s = jnp.sum(p, axis=1, keepdims=True)
    gates = p / s

    gmax = jnp.max(gates, axis=1, keepdims=True)
    eidx = jnp.min(
        jnp.where(gates == gmax, col, _NUM_EXPERTS), axis=1, keepdims=True
    )
    m1 = jnp.where(col == eidx, jnp.float32(1.0), jnp.float32(0.0))

    nblk = _NUM_TOKENS // _CUMSUM_BLK
    r = lax.broadcasted_iota(jnp.int32, (_CUMSUM_BLK, _CUMSUM_BLK), 0)
    c = lax.broadcasted_iota(jnp.int32, (_CUMSUM_BLK, _CUMSUM_BLK), 1)
    ltri = jnp.where(r >= c, jnp.float32(1.0), jnp.float32(0.0))
    loc_blocks = []
    running = jnp.zeros((1, _NUM_EXPERTS), jnp.float32)
    for i in range(nblk):
        blk = m1[i * _CUMSUM_BLK:(i + 1) * _CUMSUM_BLK, :]
        within = lax.dot_general(
            ltri, blk, (((1,), (0,)), ((), ())),
            preferred_element_type=jnp.float32,
        )
        loc_blocks.append(within + running - 1.0)
        running = running + within[_CUMSUM_BLK - 1:_CUMSUM_BLK, :]
    loc = jnp.concatenate(loc_blocks, axis=0)

    loc_sel = jnp.sum(loc * m1, axis=1, keepdims=True)
    keep = loc_sel < jnp.float32(_CAPACITY)
    fsel = eidx * _CAPACITY + loc_sel.astype(jnp.int32)
    fsel_ref[...] = jnp.where(keep, fsel, jnp.int32(-1))
    gate_ref[...] = jnp.where(keep, gmax, jnp.float32(0.0))

    sg = jnp.sum(gates, axis=0, keepdims=True)
    sm = jnp.sum(m1, axis=0, keepdims=True)
    scale = (_NUM_EXPERTS * _NUM_EXPERTS) / (
        (_NUM_EXPERTS // 4) * float(_NUM_TOKENS) * float(_NUM_TOKENS)
    )
    laux_ref[...] = jnp.sum(sg * sm, axis=1, keepdims=True) * jnp.float32(scale)


def _write_kernel(fsel_ref, gate_ref, comb_ref):
    f = fsel_ref[...]
    g = gate_ref[...]
    shp = (_WRITE_BLK, _NUM_EXPERTS, _CAPACITY)
    ei = lax.broadcasted_iota(jnp.int32, shp, 1)
    ci = lax.broadcasted_iota(jnp.int32, shp, 2)
    fi = ei * _CAPACITY + ci
    comb_ref[...] = jnp.where(fi == f, g, jnp.float32(0.0))


def _sc_dispatch_body(fsel_hbm, pat_hbm, zsrc_hbm, out_hbm, zrow, fselv,
                      pativ, dstv, rowbuf, semz, semg, sems):
    w = lax.axis_index("s") * 2 + lax.axis_index("c")

    pltpu.sync_copy(zsrc_hbm, zrow)

    rbase = w * _RPW
    for k in range(_RPW // _ZROWS):
        pltpu.async_copy(
            zrow, out_hbm.at[pl.ds(rbase + k * _ZROWS, _ZROWS)], semz
        )

    tbase = w * _TPW
    pltpu.sync_copy(fsel_hbm.at[pl.ds(tbase, _TPW)], fselv)
    for k in range(_TPW // 16):
        f = fselv[pl.ds(k * 16, 16)]
        kept = f >= 0
        cpos = f & (_CAPACITY - 1)
        epos = f >> 8
        pativ[pl.ds(k * 16, 16)] = jnp.where(kept, cpos, _CAPACITY)
        tok = tbase + k * 16 + lax.iota(jnp.int32, 16)
        dstv[pl.ds(k * 16, 16)] = (
            tok * _NUM_EXPERTS + jnp.where(kept, epos, 0)
        )

    pltpu.async_copy(pat_hbm.at[pativ], rowbuf, semg).wait()

    for k in range(_RPW // _ZROWS):
        pltpu.make_async_copy(
            zrow, out_hbm.at[pl.ds(rbase + k * _ZROWS, _ZROWS)], semz
        ).wait()
    pltpu.async_copy(rowbuf, out_hbm.at[dstv], sems).wait()


def _sc_dispatch(fsel1d, pat, zsrc):
    mesh = plsc.VectorSubcoreMesh(core_axis_name="c", subcore_axis_name="s")
    return pl.kernel(
        _sc_dispatch_body,
        out_type=jax.ShapeDtypeStruct((_ROWS, _CAPACITY), jnp.bool_),
        mesh=mesh,
        scratch_types=[
            pltpu.VMEM((_ZROWS, _CAPACITY), jnp.bool_),
            pltpu.VMEM((_TPW,), jnp.int32),
            pltpu.VMEM((_TPW,), jnp.int32),
            pltpu.VMEM((_TPW,), jnp.int32),
            pltpu.VMEM((_TPW, _CAPACITY), jnp.bool_),
            pltpu.SemaphoreType.DMA,
            pltpu.SemaphoreType.DMA,
            pltpu.SemaphoreType.DMA,
        ],
    )(fsel1d, pat, zsrc)


def kernel(input2, W2):
    fsel, gate, laux = pl.pallas_call(
        _routing_kernel,
        out_shape=[
            jax.ShapeDtypeStruct((_NUM_TOKENS, 1), jnp.int32),
            jax.ShapeDtypeStruct((_NUM_TOKENS, 1), jnp.float32),
            jax.ShapeDtypeStruct((1, 1), jnp.float32),
        ],
    )(input2, W2)

    fsel3 = fsel.reshape(_NUM_TOKENS, 1, 1)
    gate3 = gate.reshape(_NUM_TOKENS, 1, 1)
    nblk = _NUM_TOKENS // _WRITE_BLK
    combine = pl.pallas_call(
        _write_kernel,
        grid=(nblk,),
        in_specs=[
            pl.BlockSpec((_WRITE_BLK, 1, 1), lambda i: (i, 0, 0)),
            pl.BlockSpec((_WRITE_BLK, 1, 1), lambda i: (i, 0, 0)),
        ],
        out_specs=[
            pl.BlockSpec((_WRITE_BLK, _NUM_EXPERTS, _CAPACITY),
                         lambda i: (i, 0, 0)),
        ],
        out_shape=[
            jax.ShapeDtypeStruct((_NUM_TOKENS, _NUM_EXPERTS, _CAPACITY),
                                 jnp.float32),
        ],
    )(fsel3, gate3)[0]

    pat = jnp.concatenate(
        [jnp.eye(_CAPACITY, dtype=jnp.bool_),
         jnp.zeros((8, _CAPACITY), jnp.bool_)], axis=0
    )
    zsrc = jnp.zeros((_ZROWS, _CAPACITY), jnp.bool_)
    dispatch = _sc_dispatch(fsel.reshape(_NUM_TOKENS), pat, zsrc)
    dispatch = dispatch.reshape(_NUM_TOKENS, _NUM_EXPERTS, _CAPACITY)

    return (laux.reshape(()), combine, dispatch)


# speedup vs baseline: 1.2621x; 1.2621x over previous
import math

import jax
import jax.numpy as jnp
from jax import lax
from jax.experimental import pallas as pl

_NUM_TOKENS = 2048
_MODEL_DIM = 1024
_NUM_EXPERTS = 64
_CAPACITY = int(2 * math.ceil(_NUM_TOKENS / (_NUM_EXPERTS // 4)) * 1.0)
_CUMSUM_BLK = 128
_WRITE_BLK = 128


def _routing_kernel(x_ref, w_ref, fsel_ref, gate_ref, laux_ref):
    x = x_ref[...]
    w = w_ref[...]
    logits = lax.dot_general(
        x, w, (((1,), (1,)), ((), ())), preferred_element_type=jnp.float32
    )
    col = lax.broadcasted_iota(jnp.int32, (_NUM_TOKENS, _NUM_EXPERTS), 1)
    logits = jnp.where(col == 0, jnp.float32(-1000000000.0), logits)
    m = jnp.max(logits, axis=1, keepdims=True)
    p = jnp.exp(logits - m)
    s = jnp.sum(p, axis=1, keepdims=True)
    gates = p / s

    gmax = jnp.max(gates, axis=1, keepdims=True)
    eidx = jnp.min(
        jnp.where(gates == gmax, col, _NUM_EXPERTS), axis=1, keepdims=True
    )
    m1 = jnp.where(col == eidx, jnp.float32(1.0), jnp.float32(0.0))

    nblk = _NUM_TOKENS // _CUMSUM_BLK
    r = lax.broadcasted_iota(jnp.int32, (_CUMSUM_BLK, _CUMSUM_BLK), 0)
    c = lax.broadcasted_iota(jnp.int32, (_CUMSUM_BLK, _CUMSUM_BLK), 1)
    ltri = jnp.where(r >= c, jnp.float32(1.0), jnp.float32(0.0))
    loc_blocks = []
    running = jnp.zeros((1, _NUM_EXPERTS), jnp.float32)
    for i in range(nblk):
        blk = m1[i * _CUMSUM_BLK:(i + 1) * _CUMSUM_BLK, :]
        within = lax.dot_general(
            ltri, blk, (((1,), (0,)), ((), ())),
            preferred_element_type=jnp.float32,
        )
        loc_blocks.append(within + running - 1.0)
        running = running + within[_CUMSUM_BLK - 1:_CUMSUM_BLK, :]
    loc = jnp.concatenate(loc_blocks, axis=0)

    loc_sel = jnp.sum(loc * m1, axis=1, keepdims=True)
    keep = loc_sel < jnp.float32(_CAPACITY)
    fsel = eidx * _CAPACITY + loc_sel.astype(jnp.int32)
    fsel_ref[...] = jnp.where(keep, fsel, jnp.int32(-1))
    gate_ref[...] = jnp.where(keep, gmax, jnp.float32(0.0))

    sg = jnp.sum(gates, axis=0, keepdims=True)
    sm = jnp.sum(m1, axis=0, keepdims=True)
    scale = (_NUM_EXPERTS * _NUM_EXPERTS) / (
        (_NUM_EXPERTS // 4) * float(_NUM_TOKENS) * float(_NUM_TOKENS)
    )
    laux_ref[...] = jnp.sum(sg * sm, axis=1, keepdims=True) * jnp.float32(scale)


def _write_kernel(fsel_ref, gate_ref, comb_ref, disp_ref):
    f = fsel_ref[...]
    g = gate_ref[...]
    shp = (_WRITE_BLK, _NUM_EXPERTS, _CAPACITY)
    ei = lax.broadcasted_iota(jnp.int32, shp, 1)
    ci = lax.broadcasted_iota(jnp.int32, shp, 2)
    fi = ei * _CAPACITY + ci
    sel = fi == f
    comb_ref[...] = jnp.where(sel, g, jnp.float32(0.0))
    disp_ref[...] = sel


def kernel(input2, W2):
    fsel, gate, laux = pl.pallas_call(
        _routing_kernel,
        out_shape=[
            jax.ShapeDtypeStruct((_NUM_TOKENS, 1), jnp.int32),
            jax.ShapeDtypeStruct((_NUM_TOKENS, 1), jnp.float32),
            jax.ShapeDtypeStruct((1, 1), jnp.float32),
        ],
    )(input2, W2)

    fsel3 = fsel.reshape(_NUM_TOKENS, 1, 1)
    gate3 = gate.reshape(_NUM_TOKENS, 1, 1)
    nblk = _NUM_TOKENS // _WRITE_BLK
    combine, dispatch = pl.pallas_call(
        _write_kernel,
        grid=(nblk,),
        in_specs=[
            pl.BlockSpec((_WRITE_BLK, 1, 1), lambda i: (i, 0, 0)),
            pl.BlockSpec((_WRITE_BLK, 1, 1), lambda i: (i, 0, 0)),
        ],
        out_specs=[
            pl.BlockSpec((_WRITE_BLK, _NUM_EXPERTS, _CAPACITY),
                         lambda i: (i, 0, 0)),
            pl.BlockSpec((_WRITE_BLK, _NUM_EXPERTS, _CAPACITY),
                         lambda i: (i, 0, 0)),
        ],
        out_shape=[
            jax.ShapeDtypeStruct((_NUM_TOKENS, _NUM_EXPERTS, _CAPACITY),
                                 jnp.float32),
            jax.ShapeDtypeStruct((_NUM_TOKENS, _NUM_EXPERTS, _CAPACITY),
                                 jnp.bool_),
        ],
    )(fsel3, gate3)

    return (laux.reshape(()), combine, dispatch)
